# baseline (speedup 1.0000x reference)
import functools

import jax
import jax.numpy as jnp
import numpy as np
from jax.experimental import pallas as pl
from jax.experimental.pallas import tpu as pltpu

_R, _C = 64, 100000
_BLK = 1024
_CHUNK = 128
_NBLK = (_C + _BLK - 1) // _BLK
_NCHUNK = _BLK // _CHUNK
_MASK_FROM = (_C - (_NBLK - 1) * _BLK) // _CHUNK

_KS0 = 0
_KS1 = 42
_KS2 = (0x1BD11BDA ^ _KS0 ^ _KS1) & 0xFFFFFFFF
_TINY = np.float32(np.finfo(np.float32).tiny)
_NEG_INF = np.float32(-np.inf)
_ONE_BITS = np.uint32(0x3F800000)
_INT_MAX = np.int32(np.iinfo(np.int32).max)


def _rotl(x, d):
    return jax.lax.shift_left(x, np.uint32(d)) | jax.lax.shift_right_logical(
        x, np.uint32(32 - d)
    )


def _round(x0, x1, r):
    x0 = x0 + x1
    x1 = _rotl(x1, r)
    return x0, x1 ^ x0


def _threefry_bits(x1):
    x0 = x1
    x1 = _rotl(x1, 13) ^ x1
    x0, x1 = _round(x0, x1, 15)
    x0, x1 = _round(x0, x1, 26)
    x0, x1 = _round(x0, x1, 6)
    x0 = x0 + np.uint32(_KS1)
    x1 = x1 + np.uint32(_KS2 + 1)
    for r in (17, 29, 16, 24):
        x0, x1 = _round(x0, x1, r)
    x0 = x0 + np.uint32(_KS2)
    x1 = x1 + np.uint32(_KS0 + 2)
    for r in (13, 15, 26, 6):
        x0, x1 = _round(x0, x1, r)
    x1 = x1 + np.uint32(_KS1 + 3)
    for r in (17, 29, 16, 24):
        x0, x1 = _round(x0, x1, r)
    x0 = x0 + np.uint32(_KS1)
    x1 = x1 + np.uint32(_KS2 + 4)
    for r in (13, 15, 26, 6):
        x0, x1 = _round(x0, x1, r)
    x0 = x0 + np.uint32(_KS2)
    x1 = x1 + np.uint32(_KS0 + 5)
    return x0 ^ x1


def _kernel(x_ref, o_ref, vmax_ref, vidx_ref):
    j = pl.program_id(0)

    @pl.when(j == 0)
    def _init():
        vmax_ref[...] = jnp.full((_R, _CHUNK), _NEG_INF, jnp.float32)
        vidx_ref[...] = jnp.zeros((_R, _CHUNK), jnp.int32)

    @pl.when(j == _NBLK - 1)
    def _mask_tail():
        _TAIL = _C - (_NBLK - 1) * _BLK
        x_ref[:, _TAIL:_BLK] = jnp.full((_R, _BLK - _TAIL), _NEG_INF,
                                        jnp.float32)

    _H = _R // 2
    row_iota = jax.lax.broadcasted_iota(jnp.uint32, (_H, _CHUNK), 0)
    lane_u32 = jax.lax.broadcasted_iota(jnp.uint32, (_H, _CHUNK), 1)
    lane_h = jax.lax.broadcasted_iota(jnp.int32, (_H, _CHUNK), 1)
    rb = (row_iota * np.uint32(_C) + lane_u32 + np.uint32(_KS1),
          row_iota * np.uint32(_C) + lane_u32
          + np.uint32((_H * _C + _KS1) & 0xFFFFFFFF))

    vm = [vmax_ref[0:_H, :], vmax_ref[_H:_R, :]]
    vi = [vidx_ref[0:_H, :], vidx_ref[_H:_R, :]]
    for k in range(_NCHUNK):
        colbase = j * _BLK + k * _CHUNK
        for h in (0, 1):
            bits = _threefry_bits(rb[h] + colbase.astype(jnp.uint32))
            fb = jax.lax.shift_right_logical(bits, np.uint32(9)) | _ONE_BITS
            t = jax.lax.bitcast_convert_type(fb, jnp.float32) - np.float32(1.0)
            z = (x_ref[h * _H : (h + 1) * _H, k * _CHUNK : (k + 1) * _CHUNK]
                 - jnp.log(-jnp.log(t)))
            upd = z > vm[h]
            vm[h] = jnp.where(upd, z, vm[h])
            vi[h] = jnp.where(upd, j * _NCHUNK + k, vi[h])
    vmax_ref[0:_H, :] = vm[0]
    vmax_ref[_H:_R, :] = vm[1]
    vidx_ref[0:_H, :] = vi[0]
    vidx_ref[_H:_R, :] = vi[1]

    @pl.when(j == _NBLK - 1)
    def _finish():
        vmf = vmax_ref[...]
        vif = vidx_ref[...]
        lane_i32 = jax.lax.broadcasted_iota(jnp.int32, (_R, _CHUNK), 1)
        col = vif * _CHUNK + lane_i32
        m = jnp.max(vmf, axis=1, keepdims=True)
        cand = jnp.where(vmf == m, col, _INT_MAX)
        o_ref[...] = jnp.min(cand, axis=1, keepdims=True)


@functools.partial(jax.jit, static_argnames=("interpret",))
def kernel(logits, interpret=False):
    out = pl.pallas_call(
        _kernel,
        grid=(_NBLK,),
        in_specs=[pl.BlockSpec((_R, _BLK), lambda j: (0, j))],
        out_specs=pl.BlockSpec((_R, 1), lambda j: (0, 0)),
        out_shape=jax.ShapeDtypeStruct((_R, 1), jnp.int32),
        scratch_shapes=[
            pltpu.VMEM((_R, _CHUNK), jnp.float32),
            pltpu.VMEM((_R, _CHUNK), jnp.int32),
        ],
        interpret=interpret,
    )(logits)
    return out.reshape(_R)


# speedup vs baseline: 1.0321x; 1.0321x over previous
import functools

import jax
import jax.numpy as jnp
import numpy as np
from jax.experimental import pallas as pl
from jax.experimental.pallas import tpu as pltpu

_R, _C = 64, 100000
_BLK = 2048
_CHUNK = 128
_NBLK = (_C + _BLK - 1) // _BLK
_NCHUNK = _BLK // _CHUNK

_KS0 = 0
_KS1 = 42
_KS2 = (0x1BD11BDA ^ _KS0 ^ _KS1) & 0xFFFFFFFF
_NEG_INF = np.float32(-np.inf)
_ONE_BITS = np.uint32(0x3F800000)
_INT_MAX = np.int32(np.iinfo(np.int32).max)


def _rotl(x, d):
    return jax.lax.shift_left(x, np.uint32(d)) | jax.lax.shift_right_logical(
        x, np.uint32(32 - d)
    )


def _round(x0, x1, r):
    x0 = x0 + x1
    x1 = _rotl(x1, r)
    return x0, x1 ^ x0


def _threefry_bits(x1):
    x0 = x1
    x1 = _rotl(x1, 13) ^ x1
    x0, x1 = _round(x0, x1, 15)
    x0, x1 = _round(x0, x1, 26)
    x0, x1 = _round(x0, x1, 6)
    x0 = x0 + np.uint32(_KS1)
    x1 = x1 + np.uint32(_KS2 + 1)
    for r in (17, 29, 16, 24):
        x0, x1 = _round(x0, x1, r)
    x0 = x0 + np.uint32(_KS2)
    x1 = x1 + np.uint32(_KS0 + 2)
    for r in (13, 15, 26, 6):
        x0, x1 = _round(x0, x1, r)
    x1 = x1 + np.uint32(_KS1 + 3)
    for r in (17, 29, 16, 24):
        x0, x1 = _round(x0, x1, r)
    x0 = x0 + np.uint32(_KS1)
    x1 = x1 + np.uint32(_KS2 + 4)
    for r in (13, 15, 26, 6):
        x0, x1 = _round(x0, x1, r)
    x0 = x0 + np.uint32(_KS2)
    x1 = x1 + np.uint32(_KS0 + 5)
    return x0 ^ x1


def _kernel(x_ref, o_ref, vmax_ref, vidx_ref):
    j = pl.program_id(0)

    @pl.when(j == 0)
    def _init():
        vmax_ref[...] = jnp.full((_R, _CHUNK), _NEG_INF, jnp.float32)
        vidx_ref[...] = jnp.zeros((_R, _CHUNK), jnp.int32)

    @pl.when(j == _NBLK - 1)
    def _mask_tail():
        _TAIL = _C - (_NBLK - 1) * _BLK
        x_ref[:, _TAIL:_BLK] = jnp.full((_R, _BLK - _TAIL), _NEG_INF,
                                        jnp.float32)

    _H = _R // 2
    row_iota = jax.lax.broadcasted_iota(jnp.uint32, (_H, _CHUNK), 0)
    lane_u32 = jax.lax.broadcasted_iota(jnp.uint32, (_H, _CHUNK), 1)
    rb = (row_iota * np.uint32(_C) + lane_u32 + np.uint32(_KS1),
          row_iota * np.uint32(_C) + lane_u32
          + np.uint32((_H * _C + _KS1) & 0xFFFFFFFF))

    vm = [vmax_ref[0:_H, :], vmax_ref[_H:_R, :]]
    vi = [vidx_ref[0:_H, :], vidx_ref[_H:_R, :]]
    for k in range(_NCHUNK):
        colbase = j * _BLK + k * _CHUNK
        for h in (0, 1):
            bits = _threefry_bits(rb[h] + colbase.astype(jnp.uint32))
            fb = jax.lax.shift_right_logical(bits, np.uint32(9)) | _ONE_BITS
            t = jax.lax.bitcast_convert_type(fb, jnp.float32) - np.float32(1.0)
            z = (x_ref[h * _H : (h + 1) * _H, k * _CHUNK : (k + 1) * _CHUNK]
                 - jnp.log(-jnp.log(t)))
            upd = z > vm[h]
            vm[h] = jnp.where(upd, z, vm[h])
            vi[h] = jnp.where(upd, j * _NCHUNK + k, vi[h])
    vmax_ref[0:_H, :] = vm[0]
    vmax_ref[_H:_R, :] = vm[1]
    vidx_ref[0:_H, :] = vi[0]
    vidx_ref[_H:_R, :] = vi[1]

    @pl.when(j == _NBLK - 1)
    def _finish():
        vmf = vmax_ref[...]
        vif = vidx_ref[...]
        lane_i32 = jax.lax.broadcasted_iota(jnp.int32, (_R, _CHUNK), 1)
        col = vif * _CHUNK + lane_i32
        m = jnp.max(vmf, axis=1, keepdims=True)
        cand = jnp.where(vmf == m, col, _INT_MAX)
        o_ref[...] = jnp.min(cand, axis=1, keepdims=True)


@functools.partial(jax.jit, static_argnames=("interpret",))
def kernel(logits, interpret=False):
    out = pl.pallas_call(
        _kernel,
        grid=(_NBLK,),
        in_specs=[pl.BlockSpec((_R, _BLK), lambda j: (0, j))],
        out_specs=pl.BlockSpec((_R, 1), lambda j: (0, 0)),
        out_shape=jax.ShapeDtypeStruct((_R, 1), jnp.int32),
        scratch_shapes=[
            pltpu.VMEM((_R, _CHUNK), jnp.float32),
            pltpu.VMEM((_R, _CHUNK), jnp.int32),
        ],
        interpret=interpret,
    )(logits)
    return out.reshape(_R)
